# baseline (speedup 1.0000x reference)
import functools

import jax
import jax.numpy as jnp
from jax import lax
from jax.experimental import pallas as pl
from jax.experimental.pallas import tpu as pltpu
from jax.experimental.pallas import tpu_sc as plsc

_NW = 32
_BBLK = 128
_NBUF = 5
_L = 16
_PADW = 65


@functools.partial(jax.jit, static_argnums=(2, 3, 4))
def _emb_lookup_t(idx, table, nb, hist, d):
    mesh = plsc.VectorSubcoreMesh(core_axis_name="c", subcore_axis_name="s")

    @functools.partial(
        pl.kernel,
        out_type=jax.ShapeDtypeStruct((hist, d, nb), jnp.float32),
        mesh=mesh,
        scratch_types=[
            pltpu.VMEM((_BBLK, hist), jnp.int32),
            pltpu.VMEM((hist, _BBLK), jnp.int32),
            pltpu.VMEM((_NBUF, _BBLK, d), jnp.float32),
            pltpu.VMEM((_NBUF, d, _PADW * 2 - 1), jnp.float32),
            [pltpu.SemaphoreType.DMA] * _NBUF,
            [pltpu.SemaphoreType.DMA] * _NBUF,
        ],
        compiler_params=pltpu.CompilerParams(
            use_tc_tiling_on_sc=False, needs_layout_passes=False
        ),
    )
    def emb(table_hbm, idx_hbm, out_hbm, idx_raw_v, idx_v, rows_v, trans_v,
            gsems, wsems):
        wid = lax.axis_index("s") * 2 + lax.axis_index("c")
        b0 = wid * _BBLK
        pltpu.sync_copy(idx_hbm.at[pl.ds(b0, _BBLK)], idx_raw_v)
        rowvs0 = [_L * c + lax.iota(jnp.int32, _L) for c in range(_BBLK // _L)]
        for h in range(hist):
            colv = jnp.full((_L,), h, jnp.int32)
            for c in range(_BBLK // _L):
                v = plsc.load_gather(idx_raw_v, [rowvs0[c], colv])
                idx_v[h, pl.ds(_L * c, _L)] = v

        def start_gather(h, b):
            pltpu.async_copy(table_hbm.at[idx_v.at[h]], rows_v.at[b], gsems[b])

        def wait_gather(h, b):
            pltpu.make_async_copy(
                table_hbm.at[idx_v.at[h]], rows_v.at[b], gsems[b]
            ).wait()

        def write(h, b):
            return pltpu.make_async_copy(
                trans_v.at[b, :, pl.ds(0, _BBLK)],
                out_hbm.at[h, :, pl.ds(b0, _BBLK)],
                wsems[b],
            )

        def transpose(b):
            colvs = [_L * k + lax.iota(jnp.int32, _L) for k in range(d // _L)]
            for i in range(_BBLK):
                rowv = jnp.full((_L,), i, jnp.int32)
                for k in range(d // _L):
                    v = rows_v[b, i, pl.ds(_L * k, _L)]
                    plsc.store_scatter(trans_v.at[b], [colvs[k], rowv], v)

        for b in range(_NBUF):
            start_gather(b, b)

        n_groups = hist // _NBUF

        def group(g, carry):
            h0 = g * _NBUF
            for b in range(_NBUF):
                h = h0 + b
                wait_gather(h, b)

                @pl.when(g > 0)
                def _():
                    write(h - _NBUF, b).wait()

                transpose(b)
                write(h, b).start()

                @pl.when(g < n_groups - 1)
                def _():
                    start_gather(h + _NBUF, b)

            return carry

        lax.fori_loop(0, n_groups, group, None)
        for b in range(_NBUF):
            write(hist - _NBUF + b, b).wait()

    return emb(table, idx)


def kernel(indices, table):
    nb, hist = indices.shape
    _, d = table.shape
    assert nb % (_NW * _BBLK) == 0 or nb == _NW * _BBLK
    out_t = _emb_lookup_t(indices, table, nb, hist, d)
    return jnp.transpose(out_t, (2, 0, 1))


# speedup vs baseline: 1.1827x; 1.1827x over previous
import functools

import jax
import jax.numpy as jnp
from jax import lax
from jax.experimental import pallas as pl
from jax.experimental.pallas import tpu as pltpu
from jax.experimental.pallas import tpu_sc as plsc

_NW = 32
_BBLK = 128
_NBUF = 5
_NSLOT = 2 * _NBUF


@functools.partial(jax.jit, static_argnums=(2, 3, 4))
def _emb_lookup_t(idx_t, table, nb, hist, d):
    mesh = plsc.VectorSubcoreMesh(core_axis_name="c", subcore_axis_name="s")

    @functools.partial(
        pl.kernel,
        out_type=jax.ShapeDtypeStruct((hist, nb, d), jnp.float32),
        mesh=mesh,
        scratch_types=[
            pltpu.VMEM((hist, _BBLK), jnp.int32),
            pltpu.VMEM((_NSLOT, _BBLK, d), jnp.float32),
            [pltpu.SemaphoreType.DMA] * _NSLOT,
            [pltpu.SemaphoreType.DMA] * _NSLOT,
        ],
        compiler_params=pltpu.CompilerParams(use_tc_tiling_on_sc=False),
    )
    def emb(table_hbm, idx_hbm, out_hbm, idx_v, rows_v, gsems, wsems):
        wid = lax.axis_index("s") * 2 + lax.axis_index("c")
        b0 = wid * _BBLK
        pltpu.sync_copy(idx_hbm.at[:, pl.ds(b0, _BBLK)], idx_v)

        def start_gather(h, b):
            pltpu.async_copy(table_hbm.at[idx_v.at[h]], rows_v.at[b], gsems[b])

        def wait_gather(h, b):
            pltpu.make_async_copy(
                table_hbm.at[idx_v.at[h]], rows_v.at[b], gsems[b]
            ).wait()

        def write(h, b):
            return pltpu.make_async_copy(
                rows_v.at[b],
                out_hbm.at[h, pl.ds(b0, _BBLK), :],
                wsems[b],
            )

        for h in range(min(_NBUF, hist)):
            start_gather(h, h % _NSLOT)

        for h in range(hist):
            bb = h % _NSLOT
            wait_gather(h, bb)
            write(h, bb).start()
            hn = h + _NBUF
            if hn < hist:
                bb2 = hn % _NSLOT
                if h >= _NBUF:
                    write(h - _NBUF, bb2).wait()
                start_gather(hn, bb2)

        for h in range(max(0, hist - _NSLOT), hist):
            write(h, h % _NSLOT).wait()

    return emb(table, idx_t)


def kernel(indices, table):
    nb, hist = indices.shape
    _, d = table.shape
    assert nb % (_NW * _BBLK) == 0 or nb == _NW * _BBLK
    out_t = _emb_lookup_t(indices.T, table, nb, hist, d)
    return jnp.transpose(out_t, (1, 0, 2))
